# Initial kernel scaffold; baseline (speedup 1.0000x reference)
import jax
import jax.numpy as jnp
from jax.experimental import pallas as pl


def kernel(x, t, fc_w, ln_scale, ln_bias, f_w1, f_b1, f_w2, f_b2, w_gate, e_w1, e_b1, e_w2, e_b2, task_index):
    raise NotImplementedError("write your pallas kernel here")


# speedup vs baseline: 1.1414x; 1.1414x over previous
import functools

import jax
import jax.numpy as jnp
from jax.experimental import pallas as pl
from jax.experimental.pallas import tpu as pltpu

C = 768
R = 16
E = 4
HID = C // 2
NB = 512


def _se_kernel(x_ref, t_ref, fw1_ref, fb1_ref, fw2_ref, fb2_ref, a_ref):
    n = x_ref.shape[2]
    s = (jnp.sum(x_ref[0], axis=1, keepdims=True)
         + jnp.sum(t_ref[0], axis=1, keepdims=True)) * (1.0 / n)
    h = jnp.maximum(jnp.dot(fw1_ref[...], s, preferred_element_type=jnp.float32)
                    + fb1_ref[...], 0.0)
    p = jnp.dot(fw2_ref[...], h, preferred_element_type=jnp.float32) + fb2_ref[...]
    a_ref[0] = jax.nn.sigmoid(p)


def _main_kernel(x_ref, t_ref, wx_ref, wt_ref, lns_ref, lnb_ref, wgt_ref,
                 a_ref, ew1_ref, eb1_ref, ew2_ref, eb2_ref,
                 out_ref, aux_ref, imp_ref, load_ref):
    b = pl.program_id(0)
    nt = pl.program_id(1)
    nb = x_ref.shape[2]

    @pl.when((b == 0) & (nt == 0))
    def _init():
        imp_ref[...] = jnp.zeros_like(imp_ref)
        load_ref[...] = jnp.zeros_like(load_ref)

    xf = x_ref[0]
    tf = t_ref[0]
    xb = xf.astype(jnp.bfloat16)
    tb = tf.astype(jnp.bfloat16)

    y = (jnp.dot(wx_ref[...], xb, preferred_element_type=jnp.float32)
         + jnp.dot(wt_ref[...], tb, preferred_element_type=jnp.float32))
    y = jnp.maximum(y, 0.0)

    mu = jnp.mean(y, axis=0, keepdims=True)
    d = y - mu
    var = jnp.mean(d * d, axis=0, keepdims=True)
    y = d / jnp.sqrt(var + 1e-5) * lns_ref[...] + lnb_ref[...]
    yb = y.astype(jnp.bfloat16)

    lg = jnp.dot(wgt_ref[...], yb, preferred_element_type=jnp.float32)
    ii = jax.lax.broadcasted_iota(jnp.int32, lg.shape, 0)
    m1 = jnp.max(lg, axis=0, keepdims=True)
    i1 = jnp.min(jnp.where(lg == m1, ii, E), axis=0, keepdims=True)
    one1 = ii == i1
    lg2 = jnp.where(one1, -jnp.inf, lg)
    m2 = jnp.max(lg2, axis=0, keepdims=True)
    i2 = jnp.min(jnp.where(lg2 == m2, ii, E), axis=0, keepdims=True)
    one2 = ii == i2
    e21 = jnp.exp(m2 - m1)
    g1 = 1.0 / (1.0 + e21)
    g2 = e21 * g1
    gates = jnp.where(one1, g1, 0.0) + jnp.where(one2, g2, 0.0)

    imp_ref[...] += jnp.sum(gates, axis=1, keepdims=True)
    load_ref[...] += jnp.sum((gates > 0.0).astype(jnp.float32), axis=1,
                             keepdims=True)

    acc = jnp.zeros((C, nb), jnp.float32)
    for e in range(E):
        h = jnp.dot(ew1_ref[e], yb, preferred_element_type=jnp.float32)
        h = jnp.maximum(h + eb1_ref[e][:, None], 0.0)
        hb = h.astype(jnp.bfloat16)
        eo = jnp.dot(ew2_ref[e], hb, preferred_element_type=jnp.float32)
        eo = eo + eb2_ref[e][:, None]
        acc = acc + gates[e:e + 1, :] * eo

    a = a_ref[0]
    out_ref[0] = acc + xf * a + tf * (1.0 - a)

    @pl.when((b == pl.num_programs(0) - 1) & (nt == pl.num_programs(1) - 1))
    def _fin():
        imp = imp_ref[...]
        mi = jnp.mean(imp)
        vi = jnp.mean((imp - mi) ** 2)
        ld = load_ref[...]
        ml = jnp.mean(ld)
        vl = jnp.mean((ld - ml) ** 2)
        aux = (vi / (mi * mi + 1e-10) + vl / (ml * ml + 1e-10)) * 1e-2
        aux_ref[...] = jnp.reshape(aux, (1, 1))


@functools.partial(jax.jit, static_argnames=())
def kernel(x, t, fc_w, ln_scale, ln_bias, f_w1, f_b1, f_w2, f_b2,
           w_gate, e_w1, e_b1, e_w2, e_b2, task_index):
    B, Cx, H, W = x.shape
    N = H * W
    xf = x.reshape(B, Cx, N)
    tf = t.reshape(B, Cx, N)

    a = pl.pallas_call(
        _se_kernel,
        grid=(B,),
        in_specs=[
            pl.BlockSpec((1, Cx, N), lambda b: (b, 0, 0)),
            pl.BlockSpec((1, Cx, N), lambda b: (b, 0, 0)),
            pl.BlockSpec((Cx // R, Cx), lambda b: (0, 0)),
            pl.BlockSpec((Cx // R, 1), lambda b: (0, 0)),
            pl.BlockSpec((Cx, Cx // R), lambda b: (0, 0)),
            pl.BlockSpec((Cx, 1), lambda b: (0, 0)),
        ],
        out_specs=pl.BlockSpec((1, Cx, 1), lambda b: (b, 0, 0)),
        out_shape=jax.ShapeDtypeStruct((B, Cx, 1), jnp.float32),
    )(xf, tf, f_w1, f_b1.reshape(Cx // R, 1), f_w2, f_b2.reshape(Cx, 1))

    wx = fc_w[:, :Cx].astype(jnp.bfloat16)
    wt = fc_w[:, Cx:].astype(jnp.bfloat16)
    wgt = jax.lax.dynamic_index_in_dim(w_gate, task_index, 0,
                                       keepdims=False).T.astype(jnp.bfloat16)
    ew1 = e_w1.astype(jnp.bfloat16)
    ew2 = e_w2.astype(jnp.bfloat16)

    nt = N // NB
    out, aux = pl.pallas_call(
        _main_kernel,
        grid=(B, nt),
        in_specs=[
            pl.BlockSpec((1, Cx, NB), lambda b, n: (b, 0, n)),
            pl.BlockSpec((1, Cx, NB), lambda b, n: (b, 0, n)),
            pl.BlockSpec((Cx, Cx), lambda b, n: (0, 0)),
            pl.BlockSpec((Cx, Cx), lambda b, n: (0, 0)),
            pl.BlockSpec((Cx, 1), lambda b, n: (0, 0)),
            pl.BlockSpec((Cx, 1), lambda b, n: (0, 0)),
            pl.BlockSpec((E, Cx), lambda b, n: (0, 0)),
            pl.BlockSpec((1, Cx, 1), lambda b, n: (b, 0, 0)),
            pl.BlockSpec((E, HID, Cx), lambda b, n: (0, 0, 0)),
            pl.BlockSpec((E, HID), lambda b, n: (0, 0)),
            pl.BlockSpec((E, Cx, HID), lambda b, n: (0, 0, 0)),
            pl.BlockSpec((E, Cx), lambda b, n: (0, 0)),
        ],
        out_specs=[
            pl.BlockSpec((1, Cx, NB), lambda b, n: (b, 0, n)),
            pl.BlockSpec((1, 1), lambda b, n: (0, 0)),
        ],
        out_shape=[
            jax.ShapeDtypeStruct((B, Cx, N), jnp.float32),
            jax.ShapeDtypeStruct((1, 1), jnp.float32),
        ],
        scratch_shapes=[
            pltpu.VMEM((E, 1), jnp.float32),
            pltpu.VMEM((E, 1), jnp.float32),
        ],
    )(xf, tf, wx, wt, ln_scale.reshape(Cx, 1), ln_bias.reshape(Cx, 1), wgt,
      a, ew1, e_b1, ew2, e_b2)

    return out.reshape(B, Cx, H, W), aux.reshape(())


# speedup vs baseline: 1.3954x; 1.2226x over previous
import jax
import jax.numpy as jnp
from jax.experimental import pallas as pl
from jax.experimental.pallas import tpu as pltpu

C = 768
R = 16
E = 4
HID = C // 2


def _fused_kernel(x_ref, t_ref, fcw_ref, lns_ref, lnb_ref, wgt_ref,
                  fw1_ref, fb1_ref, fw2_ref, fb2_ref,
                  ew1_ref, eb1_ref, ew2_ref, eb2_ref,
                  out_ref, aux_ref, imp_ref, load_ref):
    b = pl.program_id(0)
    n = x_ref.shape[2]

    @pl.when(b == 0)
    def _init():
        imp_ref[...] = jnp.zeros_like(imp_ref)
        load_ref[...] = jnp.zeros_like(load_ref)

    xf = x_ref[0]
    tf = t_ref[0]

    s = (jnp.sum(xf, axis=1, keepdims=True)
         + jnp.sum(tf, axis=1, keepdims=True)) * (1.0 / n)
    hh = jnp.maximum(jnp.dot(fw1_ref[...], s,
                             preferred_element_type=jnp.float32)
                     + fb1_ref[...], 0.0)
    a = jax.nn.sigmoid(jnp.dot(fw2_ref[...], hh,
                               preferred_element_type=jnp.float32)
                       + fb2_ref[...])

    xb = xf.astype(jnp.bfloat16)
    tb = tf.astype(jnp.bfloat16)
    wx = fcw_ref[:, :C].astype(jnp.bfloat16)
    wt = fcw_ref[:, C:].astype(jnp.bfloat16)
    y = (jnp.dot(wx, xb, preferred_element_type=jnp.float32)
         + jnp.dot(wt, tb, preferred_element_type=jnp.float32))
    y = jnp.maximum(y, 0.0)

    mu = jnp.mean(y, axis=0, keepdims=True)
    d = y - mu
    var = jnp.mean(d * d, axis=0, keepdims=True)
    y = d / jnp.sqrt(var + 1e-5) * lns_ref[...] + lnb_ref[...]
    yb = y.astype(jnp.bfloat16)

    lg = jnp.dot(wgt_ref[...].astype(jnp.bfloat16), yb,
                 preferred_element_type=jnp.float32)
    ii = jax.lax.broadcasted_iota(jnp.int32, lg.shape, 0)
    m1 = jnp.max(lg, axis=0, keepdims=True)
    i1 = jnp.min(jnp.where(lg == m1, ii, E), axis=0, keepdims=True)
    one1 = ii == i1
    lg2 = jnp.where(one1, -jnp.inf, lg)
    m2 = jnp.max(lg2, axis=0, keepdims=True)
    i2 = jnp.min(jnp.where(lg2 == m2, ii, E), axis=0, keepdims=True)
    one2 = ii == i2
    e21 = jnp.exp(m2 - m1)
    g1 = 1.0 / (1.0 + e21)
    g2 = e21 * g1
    gates = jnp.where(one1, g1, 0.0) + jnp.where(one2, g2, 0.0)

    imp_ref[...] += jnp.sum(gates, axis=1, keepdims=True)
    load_ref[...] += jnp.sum((gates > 0.0).astype(jnp.float32), axis=1,
                             keepdims=True)

    acc = jnp.zeros((C, n), jnp.float32)
    for e in range(E):
        w1 = ew1_ref[e].astype(jnp.bfloat16)
        h = jnp.dot(w1, yb, preferred_element_type=jnp.float32)
        h = jnp.maximum(h + eb1_ref[e][:, None], 0.0)
        hb = h.astype(jnp.bfloat16)
        w2 = ew2_ref[e].astype(jnp.bfloat16)
        eo = jnp.dot(w2, hb, preferred_element_type=jnp.float32)
        eo = eo + eb2_ref[e][:, None]
        acc = acc + gates[e:e + 1, :] * eo

    out_ref[0] = acc + xf * a + tf * (1.0 - a)

    @pl.when(b == pl.num_programs(0) - 1)
    def _fin():
        imp = imp_ref[...]
        mi = jnp.mean(imp)
        vi = jnp.mean((imp - mi) ** 2)
        ld = load_ref[...]
        ml = jnp.mean(ld)
        vl = jnp.mean((ld - ml) ** 2)
        aux = (vi / (mi * mi + 1e-10) + vl / (ml * ml + 1e-10)) * 1e-2
        aux_ref[...] = jnp.reshape(aux, (1, 1))


def kernel(x, t, fc_w, ln_scale, ln_bias, f_w1, f_b1, f_w2, f_b2,
           w_gate, e_w1, e_b1, e_w2, e_b2, task_index):
    B, Cx, H, W = x.shape
    N = H * W
    xf = x.reshape(B, Cx, N)
    tf = t.reshape(B, Cx, N)
    wgt = jax.lax.dynamic_index_in_dim(w_gate, task_index, 0,
                                       keepdims=False).T

    out, aux = pl.pallas_call(
        _fused_kernel,
        grid=(B,),
        in_specs=[
            pl.BlockSpec((1, Cx, N), lambda b: (b, 0, 0)),
            pl.BlockSpec((1, Cx, N), lambda b: (b, 0, 0)),
            pl.BlockSpec((Cx, 2 * Cx), lambda b: (0, 0)),
            pl.BlockSpec((Cx, 1), lambda b: (0, 0)),
            pl.BlockSpec((Cx, 1), lambda b: (0, 0)),
            pl.BlockSpec((E, Cx), lambda b: (0, 0)),
            pl.BlockSpec((Cx // R, Cx), lambda b: (0, 0)),
            pl.BlockSpec((Cx // R, 1), lambda b: (0, 0)),
            pl.BlockSpec((Cx, Cx // R), lambda b: (0, 0)),
            pl.BlockSpec((Cx, 1), lambda b: (0, 0)),
            pl.BlockSpec((E, HID, Cx), lambda b: (0, 0, 0)),
            pl.BlockSpec((E, HID), lambda b: (0, 0)),
            pl.BlockSpec((E, Cx, HID), lambda b: (0, 0, 0)),
            pl.BlockSpec((E, Cx), lambda b: (0, 0)),
        ],
        out_specs=[
            pl.BlockSpec((1, Cx, N), lambda b: (b, 0, 0)),
            pl.BlockSpec((1, 1), lambda b: (0, 0)),
        ],
        out_shape=[
            jax.ShapeDtypeStruct((B, Cx, N), jnp.float32),
            jax.ShapeDtypeStruct((1, 1), jnp.float32),
        ],
        scratch_shapes=[
            pltpu.VMEM((E, 1), jnp.float32),
            pltpu.VMEM((E, 1), jnp.float32),
        ],
    )(xf, tf, fc_w, ln_scale.reshape(Cx, 1), ln_bias.reshape(Cx, 1), wgt,
      f_w1, f_b1.reshape(Cx // R, 1), f_w2, f_b2.reshape(Cx, 1),
      e_w1, e_b1, e_w2, e_b2)

    return out.reshape(B, Cx, H, W), aux.reshape(())


# speedup vs baseline: 1.9726x; 1.4137x over previous
import jax
import jax.numpy as jnp
from jax.experimental import pallas as pl
from jax.experimental.pallas import tpu as pltpu

C = 768
R = 16
E = 4
HID = C // 2

_DN_T = (((1,), (1,)), ((), ()))


def _dot_t(a, b):
    return jax.lax.dot_general(a, b, _DN_T,
                               preferred_element_type=jnp.float32)


def _fused_kernel(x_ref, t_ref, fcw_ref, lns_ref, lnb_ref, wg_ref,
                  fw1_ref, fb1_ref, fw2_ref, fb2_ref,
                  ew1_ref, eb1_ref, ew2_ref, eb2_ref,
                  out_ref, aux_ref, acc_ref):
    b = pl.program_id(0)
    n = x_ref.shape[1]

    @pl.when(b == 0)
    def _init():
        acc_ref[...] = jnp.zeros_like(acc_ref)

    x3 = x_ref[0]
    t3 = t_ref[0]

    s = (jnp.sum(x3, axis=0, keepdims=True)
         + jnp.sum(t3, axis=0, keepdims=True)) * (1.0 / n)
    hh = jnp.maximum(_dot_t(s, fw1_ref[...]) + fb1_ref[...], 0.0)
    a = jax.nn.sigmoid(_dot_t(hh, fw2_ref[...]) + fb2_ref[...])

    xb = x3.astype(jnp.bfloat16)
    tb = t3.astype(jnp.bfloat16)
    wx = fcw_ref[:, :C].astype(jnp.bfloat16)
    wt = fcw_ref[:, C:].astype(jnp.bfloat16)
    y = _dot_t(xb, wx) + _dot_t(tb, wt)
    y = jnp.maximum(y, 0.0)

    mu = jnp.mean(y, axis=1, keepdims=True)
    d = y - mu
    var = jnp.mean(d * d, axis=1, keepdims=True)
    y = d / jnp.sqrt(var + 1e-5) * lns_ref[...] + lnb_ref[...]
    yb = y.astype(jnp.bfloat16)

    lg = jnp.dot(yb, wg_ref[...].astype(jnp.bfloat16),
                 preferred_element_type=jnp.float32)
    ii = jax.lax.broadcasted_iota(jnp.int32, lg.shape, 1)
    m1 = jnp.max(lg, axis=1, keepdims=True)
    i1 = jnp.min(jnp.where(lg == m1, ii, E), axis=1, keepdims=True)
    one1 = ii == i1
    lg2 = jnp.where(one1, -jnp.inf, lg)
    m2 = jnp.max(lg2, axis=1, keepdims=True)
    i2 = jnp.min(jnp.where(lg2 == m2, ii, E), axis=1, keepdims=True)
    one2 = ii == i2
    e21 = jnp.exp(m2 - m1)
    g1 = 1.0 / (1.0 + e21)
    g2 = e21 * g1
    gates = jnp.where(one1, g1, 0.0) + jnp.where(one2, g2, 0.0)

    imp = jnp.sum(gates, axis=0, keepdims=True)
    ld = jnp.sum((gates > 0.0).astype(jnp.float32), axis=0, keepdims=True)
    acc_ref[0:1, 0:E] += imp
    acc_ref[1:2, 0:E] += ld

    acc = x3 * a + t3 * (1.0 - a)
    for e in range(E):
        w1 = ew1_ref[e].astype(jnp.bfloat16)
        h = _dot_t(yb, w1) + eb1_ref[e][None, :]
        h = jnp.maximum(h, 0.0)
        hb = h.astype(jnp.bfloat16)
        w2 = ew2_ref[e].astype(jnp.bfloat16)
        eo = _dot_t(hb, w2) + eb2_ref[e][None, :]
        acc = acc + gates[:, e:e + 1] * eo

    out_ref[0] = acc

    @pl.when(b == pl.num_programs(0) - 1)
    def _fin():
        imp = acc_ref[0:1, 0:E]
        mi = jnp.mean(imp)
        vi = jnp.mean((imp - mi) ** 2)
        ld = acc_ref[1:2, 0:E]
        ml = jnp.mean(ld)
        vl = jnp.mean((ld - ml) ** 2)
        aux = (vi / (mi * mi + 1e-10) + vl / (ml * ml + 1e-10)) * 1e-2
        aux_ref[...] = jnp.reshape(aux, (1, 1))


def kernel(x, t, fc_w, ln_scale, ln_bias, f_w1, f_b1, f_w2, f_b2,
           w_gate, e_w1, e_b1, e_w2, e_b2, task_index):
    B, Cx, H, W = x.shape
    N = H * W
    x3 = jnp.transpose(x, (0, 2, 3, 1)).reshape(B, N, Cx)
    t3 = jnp.transpose(t, (0, 2, 3, 1)).reshape(B, N, Cx)
    wg = jax.lax.dynamic_index_in_dim(w_gate, task_index, 0,
                                      keepdims=False)

    out3, aux = pl.pallas_call(
        _fused_kernel,
        grid=(B,),
        in_specs=[
            pl.BlockSpec((1, N, Cx), lambda b: (b, 0, 0)),
            pl.BlockSpec((1, N, Cx), lambda b: (b, 0, 0)),
            pl.BlockSpec((Cx, 2 * Cx), lambda b: (0, 0)),
            pl.BlockSpec((1, Cx), lambda b: (0, 0)),
            pl.BlockSpec((1, Cx), lambda b: (0, 0)),
            pl.BlockSpec((Cx, E), lambda b: (0, 0)),
            pl.BlockSpec((Cx // R, Cx), lambda b: (0, 0)),
            pl.BlockSpec((1, Cx // R), lambda b: (0, 0)),
            pl.BlockSpec((Cx, Cx // R), lambda b: (0, 0)),
            pl.BlockSpec((1, Cx), lambda b: (0, 0)),
            pl.BlockSpec((E, HID, Cx), lambda b: (0, 0, 0)),
            pl.BlockSpec((E, HID), lambda b: (0, 0)),
            pl.BlockSpec((E, Cx, HID), lambda b: (0, 0, 0)),
            pl.BlockSpec((E, Cx), lambda b: (0, 0)),
        ],
        out_specs=[
            pl.BlockSpec((1, N, Cx), lambda b: (b, 0, 0)),
            pl.BlockSpec((1, 1), lambda b: (0, 0)),
        ],
        out_shape=[
            jax.ShapeDtypeStruct((B, N, Cx), jnp.float32),
            jax.ShapeDtypeStruct((1, 1), jnp.float32),
        ],
        scratch_shapes=[
            pltpu.VMEM((8, 128), jnp.float32),
        ],
    )(x3, t3, fc_w, ln_scale.reshape(1, Cx), ln_bias.reshape(1, Cx), wg,
      f_w1, f_b1.reshape(1, Cx // R), f_w2, f_b2.reshape(1, Cx),
      e_w1, e_b1, e_w2, e_b2)

    out = jnp.transpose(out3.reshape(B, H, W, Cx), (0, 3, 1, 2))
    return out, aux.reshape(())
